# baseline (speedup 1.0000x reference)
import functools

import jax
import jax.numpy as jnp
from jax import lax
from jax.experimental import pallas as pl
from jax.experimental.pallas import tpu as pltpu
from jax.experimental.pallas import tpu_sc as plsc

EMBED_DIM = 8
BATCH = 16384
NUM_CORES = 2
NUM_SUBCORES = 16
NUM_WORKERS = NUM_CORES * NUM_SUBCORES
B_PER_W = BATCH // NUM_WORKERS
CHUNK = 128
NCHUNK = B_PER_W // CHUNK


def _sc_embedding_gather(idx_grouped, table):
    mesh = plsc.VectorSubcoreMesh(core_axis_name="c", subcore_axis_name="s")

    @functools.partial(
        pl.kernel,
        mesh=mesh,
        out_type=jax.ShapeDtypeStruct((BATCH, EMBED_DIM), jnp.float32),
        compiler_params=pltpu.CompilerParams(use_tc_tiling_on_sc=False),
        scratch_types=[
            pltpu.VMEM((NCHUNK, CHUNK), jnp.int32),
            pltpu.VMEM((B_PER_W, EMBED_DIM), jnp.float32),
            pltpu.SemaphoreType.DMA,
        ],
    )
    def body(idx_hbm, table_hbm, out_hbm, idx_v, rows_v, sem):
        wid = lax.axis_index("s") * NUM_CORES + lax.axis_index("c")
        pltpu.sync_copy(idx_hbm.at[wid], idx_v)
        copies = [
            pltpu.async_copy(
                table_hbm.at[idx_v.at[j]],
                rows_v.at[pl.ds(j * CHUNK, CHUNK)],
                sem,
            )
            for j in range(NCHUNK)
        ]
        for c in copies:
            c.wait()
        pltpu.sync_copy(rows_v, out_hbm.at[pl.ds(wid * B_PER_W, B_PER_W)])

    return body(idx_grouped, table)


def kernel(indices, table):
    idx = indices.astype(jnp.int32).reshape(NUM_WORKERS, NCHUNK, CHUNK)
    return _sc_embedding_gather(idx, table)


# speedup vs baseline: 8.5757x; 8.5757x over previous
import functools

import jax
import jax.numpy as jnp
from jax import lax
from jax.experimental import pallas as pl
from jax.experimental.pallas import tpu as pltpu
from jax.experimental.pallas import tpu_sc as plsc

EMBED_DIM = 8
BATCH = 16384
VOCAB1 = 1000001
MAIN_LIMIT = (VOCAB1 // 128) * 128
LAST_FULL_TILE = VOCAB1 // 128 - 1
NUM_CORES = 2
NUM_SUBCORES = 16
NUM_WORKERS = NUM_CORES * NUM_SUBCORES
B_PER_W = BATCH // NUM_WORKERS
CHUNK = 32
NCHUNK = B_PER_W // CHUNK
TILE_W = 128


def _sc_embedding_gather(idx2d, table_t, tail128):
    mesh = plsc.VectorSubcoreMesh(core_axis_name="c", subcore_axis_name="s")

    @functools.partial(
        pl.kernel,
        mesh=mesh,
        out_type=jax.ShapeDtypeStruct((EMBED_DIM, BATCH), jnp.float32),
        compiler_params=pltpu.CompilerParams(needs_layout_passes=False),
        scratch_types=[
            pltpu.VMEM((B_PER_W // 128, 128), jnp.int32),
            pltpu.VMEM((2, EMBED_DIM, CHUNK * TILE_W), jnp.float32),
            pltpu.VMEM((EMBED_DIM, B_PER_W), jnp.float32),
            pltpu.VMEM((EMBED_DIM, TILE_W), jnp.float32),
            pltpu.SemaphoreType.DMA,
        ],
    )
    def body(idx_hbm, tt_hbm, tail_hbm, out_hbm, idx_v, stg_v, vals_v, tail_v, sem):
        wid = lax.axis_index("s") * NUM_CORES + lax.axis_index("c")
        pltpu.sync_copy(idx_hbm.at[pl.ds(4 * wid, 4)], idx_v)
        pltpu.sync_copy(tail_hbm, tail_v)

        def read_vec(c, g):
            return idx_v[c // 4, pl.ds((c % 4) * CHUNK + g * 16, 16)]

        def issue_chunk(c):
            p = c % 2
            for g in range(2):
                vec = read_vec(c, g)
                tl = jnp.minimum(
                    lax.shift_right_logical(vec, 7), LAST_FULL_TILE
                )
                for j in range(16):
                    t1 = tl[j]
                    pltpu.async_copy(
                        tt_hbm.at[:, pl.ds(pl.multiple_of(t1 * TILE_W, 128), TILE_W)],
                        stg_v.at[p, :, pl.ds((g * 16 + j) * TILE_W, TILE_W)],
                        sem,
                    )

        def wait_chunk():
            for _ in range(CHUNK):
                pltpu.make_async_copy(
                    tt_hbm.at[:, pl.ds(0, TILE_W)],
                    stg_v.at[0, :, pl.ds(0, TILE_W)],
                    sem,
                ).wait()

        def extract_chunk(c):
            p = c % 2
            for g in range(2):
                vec = read_vec(c, g)
                col = jnp.bitwise_and(vec, 127)
                i_vec = lax.iota(jnp.int32, 16) + g * 16
                addr = i_vec * TILE_W + col
                m = vec >= MAIN_LIMIT
                tcol = jnp.minimum(jnp.maximum(vec - MAIN_LIMIT, 0), 127)
                p_vec = jnp.full((16,), jnp.int32(0)) + p
                for d in range(EMBED_DIM):
                    row = jnp.full((16,), d, jnp.int32)
                    val = plsc.load_gather(stg_v, [p_vec, row, addr])
                    tval = plsc.load_gather(tail_v, [row, tcol])
                    vals_v[d, pl.ds(c * CHUNK + g * 16, 16)] = jnp.where(m, tval, val)

        issue_chunk(0)

        def step(t, carry):
            wait_chunk()
            issue_chunk(t)
            extract_chunk(t - 1)
            return carry

        lax.fori_loop(1, NCHUNK, step, 0)
        wait_chunk()
        extract_chunk(NCHUNK - 1)
        pltpu.sync_copy(vals_v, out_hbm.at[:, pl.ds(B_PER_W * wid, B_PER_W)])

    return body(idx2d, table_t, tail128)


def kernel(indices, table):
    idx2d = indices.astype(jnp.int32).reshape(128, 128)
    tail128 = jnp.pad(table[MAIN_LIMIT:], ((0, 128 - (VOCAB1 - MAIN_LIMIT)), (0, 0))).T
    out_t = _sc_embedding_gather(idx2d, table.T, tail128)
    return out_t.T


# speedup vs baseline: 9.1300x; 1.0646x over previous
import functools

import jax
import jax.numpy as jnp
from jax import lax
from jax.experimental import pallas as pl
from jax.experimental.pallas import tpu as pltpu
from jax.experimental.pallas import tpu_sc as plsc

EMBED_DIM = 8
BATCH = 16384
VOCAB1 = 1000001
MAIN_LIMIT = (VOCAB1 // 128) * 128
LAST_FULL_TILE = VOCAB1 // 128 - 1
NUM_CORES = 2
NUM_SUBCORES = 16
NUM_WORKERS = NUM_CORES * NUM_SUBCORES
B_PER_W = BATCH // NUM_WORKERS
CHUNK = 32
NCHUNK = B_PER_W // CHUNK
TILE_W = 128


def _sc_embedding_gather(idx2d, table_t, tail128):
    mesh = plsc.VectorSubcoreMesh(core_axis_name="c", subcore_axis_name="s")

    @functools.partial(
        pl.kernel,
        mesh=mesh,
        out_type=jax.ShapeDtypeStruct((EMBED_DIM, BATCH), jnp.float32),
        compiler_params=pltpu.CompilerParams(needs_layout_passes=False),
        scratch_types=[
            pltpu.VMEM((B_PER_W // 128, 128), jnp.int32),
            pltpu.VMEM((2, EMBED_DIM, CHUNK * TILE_W), jnp.float32),
            pltpu.VMEM((EMBED_DIM, B_PER_W), jnp.float32),
            pltpu.VMEM((EMBED_DIM, TILE_W), jnp.float32),
            pltpu.SemaphoreType.DMA,
            pltpu.SemaphoreType.DMA,
        ],
    )
    def body(idx_hbm, tt_hbm, tail_hbm, out_hbm, idx_v, stg_v, vals_v, tail_v,
             sem0, sem1):
        wid = lax.axis_index("s") * NUM_CORES + lax.axis_index("c")
        pltpu.sync_copy(idx_hbm.at[pl.ds(4 * wid, 4)], idx_v)
        pltpu.sync_copy(tail_hbm, tail_v)

        def read_vec(c, g):
            return idx_v[c // 4, pl.ds((c % 4) * CHUNK + g * 16, 16)]

        def issue_chunk(c, p, sem):
            for g in range(2):
                vec = read_vec(c, g)
                tl = jnp.minimum(
                    lax.shift_right_logical(vec, 7), LAST_FULL_TILE
                )
                for j in range(16):
                    t1 = tl[j]
                    pltpu.async_copy(
                        tt_hbm.at[:, pl.ds(pl.multiple_of(t1 * TILE_W, 128), TILE_W)],
                        stg_v.at[p, :, pl.ds((g * 16 + j) * TILE_W, TILE_W)],
                        sem,
                    )

        def wait_chunk(sem):
            for _ in range(CHUNK):
                pltpu.make_async_copy(
                    tt_hbm.at[:, pl.ds(0, TILE_W)],
                    stg_v.at[0, :, pl.ds(0, TILE_W)],
                    sem,
                ).wait()

        def extract_chunk(c, p):
            for g in range(2):
                vec = read_vec(c, g)
                col = jnp.bitwise_and(vec, 127)
                i_vec = lax.iota(jnp.int32, 16) + g * 16
                addr = i_vec * TILE_W + col
                m = vec >= MAIN_LIMIT
                tcol = jnp.minimum(jnp.maximum(vec - MAIN_LIMIT, 0), 127)
                p_vec = jnp.full((16,), p, jnp.int32)
                for d in range(EMBED_DIM):
                    row = jnp.full((16,), d, jnp.int32)
                    val = plsc.load_gather(stg_v, [p_vec, row, addr])
                    tval = plsc.load_gather(tail_v, [row, tcol])
                    vals_v[d, pl.ds(c * CHUNK + g * 16, 16)] = jnp.where(m, tval, val)

        issue_chunk(0, 0, sem0)
        issue_chunk(1, 1, sem1)

        def step(u, carry):
            c = 2 * u
            wait_chunk(sem0)
            extract_chunk(c, 0)
            issue_chunk(c + 2, 0, sem0)
            wait_chunk(sem1)
            extract_chunk(c + 1, 1)
            issue_chunk(c + 3, 1, sem1)
            return carry

        lax.fori_loop(0, NCHUNK // 2 - 1, step, 0)
        wait_chunk(sem0)
        extract_chunk(NCHUNK - 2, 0)
        wait_chunk(sem1)
        extract_chunk(NCHUNK - 1, 1)
        pltpu.sync_copy(vals_v, out_hbm.at[:, pl.ds(B_PER_W * wid, B_PER_W)])

    return body(idx2d, table_t, tail128)


def kernel(indices, table):
    idx2d = indices.astype(jnp.int32).reshape(128, 128)
    tail128 = jnp.pad(table[MAIN_LIMIT:], ((0, 128 - (VOCAB1 - MAIN_LIMIT)), (0, 0))).T
    out_t = _sc_embedding_gather(idx2d, table.T, tail128)
    return out_t.T


# speedup vs baseline: 9.2058x; 1.0083x over previous
import functools

import jax
import jax.numpy as jnp
from jax import lax
from jax.experimental import pallas as pl
from jax.experimental.pallas import tpu as pltpu
from jax.experimental.pallas import tpu_sc as plsc

EMBED_DIM = 8
BATCH = 16384
VOCAB1 = 1000001
MAIN_LIMIT = (VOCAB1 // 128) * 128
LAST_FULL_TILE = VOCAB1 // 128 - 1
NUM_CORES = 2
NUM_SUBCORES = 16
NUM_WORKERS = NUM_CORES * NUM_SUBCORES
B_PER_W = BATCH // NUM_WORKERS
CHUNK = 32
NCHUNK = B_PER_W // CHUNK
TILE_W = 128


def _sc_embedding_gather(idx2d, table_t, tail128):
    mesh = plsc.VectorSubcoreMesh(core_axis_name="c", subcore_axis_name="s")

    @functools.partial(
        pl.kernel,
        mesh=mesh,
        out_type=jax.ShapeDtypeStruct((EMBED_DIM, BATCH), jnp.float32),
        compiler_params=pltpu.CompilerParams(needs_layout_passes=False),
        scratch_types=[
            pltpu.VMEM((B_PER_W // 128, 128), jnp.int32),
            pltpu.VMEM((2, EMBED_DIM, CHUNK * TILE_W), jnp.float32),
            pltpu.VMEM((EMBED_DIM, B_PER_W), jnp.float32),
            pltpu.VMEM((EMBED_DIM, TILE_W), jnp.float32),
            pltpu.SemaphoreType.DMA,
            pltpu.SemaphoreType.DMA,
        ],
    )
    def body(idx_hbm, tt_hbm, tail_hbm, out_hbm, idx_v, stg_v, vals_v, tail_v,
             sem0, sem1):
        wid = lax.axis_index("s") * NUM_CORES + lax.axis_index("c")
        pltpu.sync_copy(idx_hbm.at[pl.ds(4 * wid, 4)], idx_v)
        pltpu.sync_copy(tail_hbm, tail_v)

        def read_vec(c, g):
            return idx_v[c // 4, pl.ds((c % 4) * CHUNK + g * 16, 16)]

        def issue_chunk(c, p, sem):
            for g in range(2):
                vec = read_vec(c, g)
                tl = jnp.minimum(
                    lax.shift_right_logical(vec, 7), LAST_FULL_TILE
                )
                for j in range(16):
                    t1 = tl[j]
                    pltpu.async_copy(
                        tt_hbm.at[:, pl.ds(pl.multiple_of(t1 * TILE_W, 128), TILE_W)],
                        stg_v.at[p, :, pl.ds((g * 16 + j) * TILE_W, TILE_W)],
                        sem,
                    )

        def wait_chunk(sem):
            pltpu.make_async_copy(
                tt_hbm.at[:, pl.ds(0, CHUNK * TILE_W)],
                stg_v.at[0],
                sem,
            ).wait()

        def extract_chunk(c, p):
            for g in range(2):
                vec = read_vec(c, g)
                col = jnp.bitwise_and(vec, 127)
                i_vec = lax.iota(jnp.int32, 16) + g * 16
                addr = i_vec * TILE_W + col
                m = vec >= MAIN_LIMIT
                tcol = jnp.minimum(jnp.maximum(vec - MAIN_LIMIT, 0), 127)
                p_vec = jnp.full((16,), p, jnp.int32)
                for d in range(EMBED_DIM):
                    row = jnp.full((16,), d, jnp.int32)
                    val = plsc.load_gather(stg_v, [p_vec, row, addr])
                    tval = plsc.load_gather(tail_v, [row, tcol])
                    vals_v[d, pl.ds(c * CHUNK + g * 16, 16)] = jnp.where(m, tval, val)

        issue_chunk(0, 0, sem0)
        issue_chunk(1, 1, sem1)

        def step(u, carry):
            c = 2 * u
            wait_chunk(sem0)
            extract_chunk(c, 0)
            issue_chunk(c + 2, 0, sem0)
            wait_chunk(sem1)
            extract_chunk(c + 1, 1)
            issue_chunk(c + 3, 1, sem1)
            return carry

        lax.fori_loop(0, NCHUNK // 2 - 1, step, 0)
        wait_chunk(sem0)
        extract_chunk(NCHUNK - 2, 0)
        wait_chunk(sem1)
        extract_chunk(NCHUNK - 1, 1)
        pltpu.sync_copy(vals_v, out_hbm.at[:, pl.ds(B_PER_W * wid, B_PER_W)])

    return body(idx2d, table_t, tail128)


def kernel(indices, table):
    idx2d = indices.astype(jnp.int32).reshape(128, 128)
    tail128 = jnp.pad(table[MAIN_LIMIT:], ((0, 128 - (VOCAB1 - MAIN_LIMIT)), (0, 0))).T
    out_t = _sc_embedding_gather(idx2d, table.T, tail128)
    return out_t.T
